# baseline (speedup 1.0000x reference)
import functools

import jax
import jax.numpy as jnp
from jax import lax
from jax.experimental import pallas as pl
from jax.experimental.pallas import tpu as pltpu
from jax.experimental.pallas import tpu_sc as plsc

NC, NS, L = 2, 16, 16
NW = NC * NS
N = 10000
NPAD = 10240
SLICE = NPAD // NS
E = 320000
EPT = E // NW
D = 128
CH = 80
NB = 25
NBLK = EPT // (NB * CH)
NBUF = 3

_mesh = plsc.VectorSubcoreMesh(core_axis_name="c", subcore_axis_name="s")


@functools.partial(
    pl.kernel,
    out_type=jax.ShapeDtypeStruct((NC, NPAD), jnp.float32),
    mesh=_mesh,
    scratch_types=[
        pltpu.VMEM((NB, CH), jnp.int32),
        pltpu.VMEM((NB, CH), jnp.int32),
        pltpu.VMEM((CH,), jnp.float32),
        pltpu.VMEM_SHARED((NPAD,), jnp.float32),
        pltpu.SemaphoreType.DMA,
        pltpu.SemaphoreType.DMA,
    ],
)
def _degree_kernel(dst_hbm, deg_out, dst_b0, dst_b1, ones_v, deg_sp,
                   sem0, sem1):
    c = lax.axis_index("c")
    s = lax.axis_index("s")
    wid = c * NS + s

    def fill(i, val):
        ones_v[pl.ds(i * L, L)] = jnp.full((L,), val, jnp.float32)
        return val
    lax.fori_loop(0, CH // L, fill, 0.0)
    for k in range(SLICE // CH):
        pltpu.sync_copy(ones_v, deg_sp.at[pl.ds(s * SLICE + k * CH, CH)])
    plsc.subcore_barrier()
    lax.fori_loop(0, CH // L, fill, 1.0)

    bufs = (dst_b0, dst_b1)
    sems = (sem0, sem1)

    def fire(buf, sem):
        def one(i, _):
            pltpu.async_copy(ones_v, deg_sp.at[buf.at[i]], sem, add=True)
            return 0
        lax.fori_loop(0, NB, one, 0)

    def drain(buf, sem):
        def one(i, _):
            pltpu.make_async_copy(ones_v, deg_sp.at[buf.at[0]], sem).wait()
            return 0
        lax.fori_loop(0, NB, one, 0)

    for blk in range(NBLK):
        p = blk % 2
        if blk >= 2:
            drain(bufs[p], sems[p])
        pltpu.sync_copy(dst_hbm.at[wid, blk], bufs[p])
        fire(bufs[p], sems[p])
    for blk in range(NBLK - 2, NBLK):
        p = blk % 2
        drain(bufs[p], sems[p])

    plsc.subcore_barrier()
    pltpu.sync_copy(deg_sp.at[pl.ds(s * SLICE, SLICE)],
                    deg_out.at[c, pl.ds(s * SLICE, SLICE)])


@functools.partial(
    pl.kernel,
    out_type=jax.ShapeDtypeStruct((NC, NPAD, D), jnp.float32),
    mesh=_mesh,
    scratch_types=[
        pltpu.VMEM((NB, CH), jnp.int32),
        pltpu.VMEM((NB, CH), jnp.int32),
        pltpu.VMEM((CH, D), jnp.float32),
        pltpu.VMEM((CH, D), jnp.float32),
        pltpu.VMEM((CH, D), jnp.float32),
        pltpu.VMEM_SHARED((NPAD, D), jnp.float32),
        pltpu.SemaphoreType.DMA,
        pltpu.SemaphoreType.DMA,
        pltpu.SemaphoreType.DMA,
        pltpu.SemaphoreType.DMA,
        pltpu.SemaphoreType.DMA,
        pltpu.SemaphoreType.DMA,
    ],
)
def _scatter_kernel(src_hbm, dst_hbm, xs_hbm, s_out,
                    src_blk, dst_blk, r0, r1, r2, agg_sp,
                    g0, g1, g2, s0, s1, s2):
    c = lax.axis_index("c")
    s = lax.axis_index("s")
    wid = c * NS + s
    rows = (r0, r1, r2)
    gsem = (g0, g1, g2)
    ssem = (s0, s1, s2)

    def zr(r, _):
        for j in range(D // L):
            r0[r, pl.ds(j * L, L)] = jnp.zeros((L,), jnp.float32)
        return 0
    lax.fori_loop(0, CH, zr, 0)
    for k in range(SLICE // CH):
        pltpu.sync_copy(r0, agg_sp.at[pl.ds(s * SLICE + k * CH, CH)])
    plsc.subcore_barrier()

    def gstart(i, k):
        pltpu.async_copy(xs_hbm.at[src_blk.at[i]], rows[k], gsem[k])

    def gwait(k):
        pltpu.make_async_copy(xs_hbm.at[src_blk.at[0]], rows[k],
                              gsem[k]).wait()

    def sstart(i, k):
        pltpu.async_copy(rows[k], agg_sp.at[dst_blk.at[i]], ssem[k],
                         add=True)

    def swait(k):
        pltpu.make_async_copy(rows[k], agg_sp.at[dst_blk.at[0]],
                              ssem[k]).wait()

    for blk in range(NBLK):
        pltpu.sync_copy(src_hbm.at[wid, blk], src_blk)
        pltpu.sync_copy(dst_hbm.at[wid, blk], dst_blk)
        for k in range(NBUF):
            gstart(k, k)

        def tri(j, _):
            for k in range(NBUF):
                i = NBUF * j + k
                gwait(k)
                sstart(i, k)
                swait(k)
                gstart(i + NBUF, k)
            return 0
        lax.fori_loop(0, 7, tri, 0)
        gwait(0)
        sstart(21, 0)
        swait(0)
        gstart(24, 0)
        for k, i in ((1, 22), (2, 23), (0, 24)):
            gwait(k)
            sstart(i, k)
            swait(k)

    plsc.subcore_barrier()
    pltpu.sync_copy(agg_sp.at[pl.ds(s * SLICE, SLICE)],
                    s_out.at[c, pl.ds(s * SLICE, SLICE)])


def _prescale_body(deg_ref, x_ref, adj_ref, xs_ref, dis_ref):
    deg = deg_ref[0] + deg_ref[1] + 1.0 + adj_ref[0, 0]
    dis = lax.rsqrt(deg)
    dis_ref[...] = dis
    xs_ref[...] = x_ref[...] * dis


def _prescale(deg2, x, adj):
    pb = 2000
    return pl.pallas_call(
        _prescale_body,
        grid=(N // pb,),
        in_specs=[
            pl.BlockSpec((NC, pb, 1), lambda i: (0, i, 0)),
            pl.BlockSpec((pb, D), lambda i: (i, 0)),
            pl.BlockSpec((1, 1), lambda i: (0, 0)),
        ],
        out_specs=[
            pl.BlockSpec((pb, D), lambda i: (i, 0)),
            pl.BlockSpec((pb, 1), lambda i: (i, 0)),
        ],
        out_shape=[
            jax.ShapeDtypeStruct((N, D), jnp.float32),
            jax.ShapeDtypeStruct((N, 1), jnp.float32),
        ],
    )(deg2, x, adj)


def _combine_body(s_ref, xs_ref, dis_ref, wt_ref, out_ref):
    agg = s_ref[0] + s_ref[1] + xs_ref[...]
    a = agg * dis_ref[...]
    out_ref[...] = jnp.dot(a, wt_ref[...], preferred_element_type=jnp.float32)


def _combine(s2, xs, dis, wt):
    rb = 5000
    grid = N // rb
    return pl.pallas_call(
        _combine_body,
        grid=(grid,),
        in_specs=[
            pl.BlockSpec((NC, rb, D), lambda i: (0, i, 0)),
            pl.BlockSpec((rb, D), lambda i: (i, 0)),
            pl.BlockSpec((rb, 1), lambda i: (i, 0)),
            pl.BlockSpec((D, D), lambda i: (0, 0)),
        ],
        out_specs=pl.BlockSpec((rb, D), lambda i: (i, 0)),
        out_shape=jax.ShapeDtypeStruct((N, D), jnp.float32),
    )(s2, xs, dis, wt)


def kernel(x, edge_index, num_nodes, W):
    src4 = edge_index[0].astype(jnp.int32).reshape(NW, NBLK, NB, CH)
    dst4 = edge_index[1].astype(jnp.int32).reshape(NW, NBLK, NB, CH)
    adj = (jnp.asarray(num_nodes, jnp.float32) - x.shape[0]).reshape(1, 1)

    deg_p = _degree_kernel(dst4)
    xs, dis = _prescale(deg_p.reshape(NC, NPAD, 1), x, adj)
    s_p = _scatter_kernel(src4, dst4, xs)
    return _combine(s_p, xs, dis, W.T)


# speedup vs baseline: 1.0349x; 1.0349x over previous
import functools

import jax
import jax.numpy as jnp
from jax import lax
from jax.experimental import pallas as pl
from jax.experimental.pallas import tpu as pltpu
from jax.experimental.pallas import tpu_sc as plsc

NC, NS, L = 2, 16, 16
NW = NC * NS
N = 10000
NPAD = 10240
SLICE = NPAD // NS
E = 320000
EPT = 10240
EP = NW * EPT
D = 128
CH = 64
NB = 40
NBLK = EPT // (NB * CH)
NBUF = 4

_mesh = plsc.VectorSubcoreMesh(core_axis_name="c", subcore_axis_name="s")


@functools.partial(
    pl.kernel,
    out_type=jax.ShapeDtypeStruct((NC, NPAD), jnp.float32),
    mesh=_mesh,
    scratch_types=[
        pltpu.VMEM((NB, CH), jnp.int32),
        pltpu.VMEM((NB, CH), jnp.int32),
        pltpu.VMEM((CH,), jnp.float32),
        pltpu.VMEM_SHARED((NPAD,), jnp.float32),
        pltpu.SemaphoreType.DMA,
        pltpu.SemaphoreType.DMA,
    ],
)
def _degree_kernel(dst_hbm, deg_out, dst_b0, dst_b1, ones_v, deg_sp,
                   sem0, sem1):
    c = lax.axis_index("c")
    s = lax.axis_index("s")
    wid = c * NS + s

    def fill(i, val):
        ones_v[pl.ds(i * L, L)] = jnp.full((L,), val, jnp.float32)
        return val
    lax.fori_loop(0, CH // L, fill, 0.0)
    for k in range(SLICE // CH):
        pltpu.sync_copy(ones_v, deg_sp.at[pl.ds(s * SLICE + k * CH, CH)])
    plsc.subcore_barrier()
    lax.fori_loop(0, CH // L, fill, 1.0)

    bufs = (dst_b0, dst_b1)
    sems = (sem0, sem1)

    def fire(buf, sem):
        def one(i, _):
            pltpu.async_copy(ones_v, deg_sp.at[buf.at[i]], sem, add=True)
            return 0
        lax.fori_loop(0, NB, one, 0)

    def drain(buf, sem):
        def one(i, _):
            pltpu.make_async_copy(ones_v, deg_sp.at[buf.at[0]], sem).wait()
            return 0
        lax.fori_loop(0, NB, one, 0)

    for blk in range(NBLK):
        p = blk % 2
        if blk >= 2:
            drain(bufs[p], sems[p])
        pltpu.sync_copy(dst_hbm.at[wid, blk], bufs[p])
        fire(bufs[p], sems[p])
    for blk in range(NBLK - 2, NBLK):
        p = blk % 2
        drain(bufs[p], sems[p])

    plsc.subcore_barrier()
    pltpu.sync_copy(deg_sp.at[pl.ds(s * SLICE, SLICE)],
                    deg_out.at[c, pl.ds(s * SLICE, SLICE)])


@functools.partial(
    pl.kernel,
    out_type=jax.ShapeDtypeStruct((NC, NPAD, D), jnp.float32),
    mesh=_mesh,
    scratch_types=[
        pltpu.VMEM((NB, CH), jnp.int32),
        pltpu.VMEM((NB, CH), jnp.int32),
        pltpu.VMEM((CH, D), jnp.float32),
        pltpu.VMEM((CH, D), jnp.float32),
        pltpu.VMEM((CH, D), jnp.float32),
        pltpu.VMEM((CH, D), jnp.float32),
        pltpu.VMEM_SHARED((NPAD, D), jnp.float32),
        pltpu.SemaphoreType.DMA,
        pltpu.SemaphoreType.DMA,
        pltpu.SemaphoreType.DMA,
        pltpu.SemaphoreType.DMA,
        pltpu.SemaphoreType.DMA,
        pltpu.SemaphoreType.DMA,
        pltpu.SemaphoreType.DMA,
        pltpu.SemaphoreType.DMA,
    ],
)
def _scatter_kernel(src_hbm, dst_hbm, xs_hbm, s_out,
                    src_blk, dst_blk, r0, r1, r2, r3, agg_sp,
                    g0, g1, g2, g3, s0, s1, s2, s3):
    c = lax.axis_index("c")
    s = lax.axis_index("s")
    wid = c * NS + s
    rows = (r0, r1, r2, r3)
    gsem = (g0, g1, g2, g3)
    ssem = (s0, s1, s2, s3)

    def zr(r, _):
        for j in range(D // L):
            r0[r, pl.ds(j * L, L)] = jnp.zeros((L,), jnp.float32)
        return 0
    lax.fori_loop(0, CH, zr, 0)
    for k in range(SLICE // CH):
        pltpu.sync_copy(r0, agg_sp.at[pl.ds(s * SLICE + k * CH, CH)])
    plsc.subcore_barrier()

    def gstart(i, k):
        pltpu.async_copy(xs_hbm.at[src_blk.at[i]], rows[k], gsem[k])

    def gwait(k):
        pltpu.make_async_copy(xs_hbm.at[src_blk.at[0]], rows[k],
                              gsem[k]).wait()

    def sstart(i, k):
        pltpu.async_copy(rows[k], agg_sp.at[dst_blk.at[i]], ssem[k],
                         add=True)

    def swait(k):
        pltpu.make_async_copy(rows[k], agg_sp.at[dst_blk.at[0]],
                              ssem[k]).wait()

    for blk in range(NBLK):
        pltpu.sync_copy(src_hbm.at[wid, blk], src_blk)
        pltpu.sync_copy(dst_hbm.at[wid, blk], dst_blk)
        for k in range(NBUF):
            gstart(k, k)

        def quad(j, _):
            for k in range(NBUF):
                i = NBUF * j + k
                gwait(k)
                sstart(i, k)
                swait(k)
                gstart(i + NBUF, k)
            return 0
        lax.fori_loop(0, (NB - NBUF) // NBUF, quad, 0)

        for k in range(NBUF):
            gwait(k)
            sstart(NB - NBUF + k, k)
        for k in range(NBUF):
            swait(k)

    plsc.subcore_barrier()
    pltpu.sync_copy(agg_sp.at[pl.ds(s * SLICE, SLICE)],
                    s_out.at[c, pl.ds(s * SLICE, SLICE)])


def _prescale_body(deg_ref, x_ref, adj_ref, xs_ref, dis_ref):
    deg = deg_ref[0] + deg_ref[1] + 1.0 + adj_ref[0, 0]
    dis = lax.rsqrt(deg)
    dis_ref[...] = dis
    xs_ref[...] = x_ref[...] * dis


def _prescale(deg2, x, adj):
    pb = 2000
    return pl.pallas_call(
        _prescale_body,
        grid=(N // pb,),
        in_specs=[
            pl.BlockSpec((NC, pb, 1), lambda i: (0, i, 0)),
            pl.BlockSpec((pb, D), lambda i: (i, 0)),
            pl.BlockSpec((1, 1), lambda i: (0, 0)),
        ],
        out_specs=[
            pl.BlockSpec((pb, D), lambda i: (i, 0)),
            pl.BlockSpec((pb, 1), lambda i: (i, 0)),
        ],
        out_shape=[
            jax.ShapeDtypeStruct((N, D), jnp.float32),
            jax.ShapeDtypeStruct((N, 1), jnp.float32),
        ],
    )(deg2, x, adj)


def _combine_body(s_ref, xs_ref, dis_ref, wt_ref, out_ref):
    agg = s_ref[0] + s_ref[1] + xs_ref[...]
    a = agg * dis_ref[...]
    out_ref[...] = jnp.dot(a, wt_ref[...], preferred_element_type=jnp.float32)


def _combine(s2, xs, dis, wt):
    rb = 5000
    grid = N // rb
    return pl.pallas_call(
        _combine_body,
        grid=(grid,),
        in_specs=[
            pl.BlockSpec((NC, rb, D), lambda i: (0, i, 0)),
            pl.BlockSpec((rb, D), lambda i: (i, 0)),
            pl.BlockSpec((rb, 1), lambda i: (i, 0)),
            pl.BlockSpec((D, D), lambda i: (0, 0)),
        ],
        out_specs=pl.BlockSpec((rb, D), lambda i: (i, 0)),
        out_shape=jax.ShapeDtypeStruct((N, D), jnp.float32),
    )(s2, xs, dis, wt)


def kernel(x, edge_index, num_nodes, W):
    pad = jnp.arange(EP - E, dtype=jnp.int32)
    srcp = jnp.concatenate([edge_index[0].astype(jnp.int32), pad % N])
    dstp = jnp.concatenate(
        [edge_index[1].astype(jnp.int32), N + pad % (NPAD - N)])
    src4 = srcp.reshape(NW, NBLK, NB, CH)
    dst4 = dstp.reshape(NW, NBLK, NB, CH)
    adj = (jnp.asarray(num_nodes, jnp.float32) - x.shape[0]).reshape(1, 1)

    deg_p = _degree_kernel(dst4)
    xs, dis = _prescale(deg_p.reshape(NC, NPAD, 1), x, adj)
    s_p = _scatter_kernel(src4, dst4, xs)
    return _combine(s_p, xs, dis, W.T)
